# baseline (speedup 1.0000x reference)
import jax
import jax.numpy as jnp
from jax import lax
from jax.experimental import pallas as pl
from jax.experimental.pallas import tpu as pltpu
from jax.experimental.pallas import tpu_sc as plsc

N = 32768
D = 768
E = 64
BLOCK_N = 8192
S = 2

NW = 32
ROWS_PER_W = N // NW
HALF = ROWS_PER_W // 2
GROUPS_H = HALF // 16


def _lex_take(v1, j1, v2, j2):
    return (v1 > v2) | ((v1 == v2) & (j1 < j2))


def _merge_lex(A, B):
    m1a, a1a, m2a, a2a = A
    m1b, a1b, m2b, a2b = B
    take = _lex_take(m1a, a1a, m1b, a1b)
    m1 = jnp.where(take, m1a, m1b)
    a1 = jnp.where(take, a1a, a1b)
    sa = _lex_take(m2a, a2a, m1b, a1b)
    sb = _lex_take(m1a, a1a, m2b, a2b)
    m2 = jnp.where(take, jnp.where(sa, m2a, m1b), jnp.where(sb, m1a, m2b))
    a2 = jnp.where(take, jnp.where(sa, a2a, a1b), jnp.where(sb, a1a, a2b))
    return m1, a1, m2, a2


def _matmul_reduce_body(x_ref, w_ref, vals_ref, idx_ref):
    lt = lax.dot_general(
        w_ref[...], x_ref[...], (((1,), (1,)), ((), ())),
        preferred_element_type=jnp.float32)
    a, b = lt[0:32], lt[32:64]
    ia = jax.lax.broadcasted_iota(jnp.int32, (32, BLOCK_N), 0)
    ib = ia + 32
    take = a >= b
    t = (jnp.maximum(a, b), jnp.where(take, ia, ib),
         jnp.minimum(a, b), jnp.where(take, ib, ia))
    for k in (16, 8, 4, 2):
        t = _merge_lex(tuple(v[:k] for v in t), tuple(v[k:] for v in t))
    m1, i1, m2, i2 = t
    vals_ref[...] = jnp.concatenate([m1, m2], axis=0)
    idx_ref[...] = jnp.concatenate([i1, i2], axis=0)


def _tc_partial_top2(x, W):
    return pl.pallas_call(
        _matmul_reduce_body,
        grid=(N // BLOCK_N,),
        in_specs=[
            pl.BlockSpec((BLOCK_N, D), lambda i: (i, 0)),
            pl.BlockSpec((E, D), lambda i: (0, 0)),
        ],
        out_specs=[
            pl.BlockSpec((2 * S, BLOCK_N), lambda i: (0, i)),
            pl.BlockSpec((2 * S, BLOCK_N), lambda i: (0, i)),
        ],
        out_shape=[
            jax.ShapeDtypeStruct((2 * S, N), jnp.float32),
            jax.ShapeDtypeStruct((2 * S, N), jnp.int32),
        ],
    )(x, W)


def _route_body(vals_hbm, idx_hbm, out_hbm,
                v_v0, v_v1, i_v0, i_v1, ob_v, sem0, sem1, sem2, sem3):
    wid = lax.axis_index("s") * 2 + lax.axis_index("c")
    base = wid * ROWS_PER_W

    cps = [
        pltpu.make_async_copy(vals_hbm.at[:, pl.ds(base, HALF)], v_v0, sem0),
        pltpu.make_async_copy(idx_hbm.at[:, pl.ds(base, HALF)], i_v0, sem1),
        pltpu.make_async_copy(vals_hbm.at[:, pl.ds(base + HALF, HALF)],
                              v_v1, sem2),
        pltpu.make_async_copy(idx_hbm.at[:, pl.ds(base + HALF, HALF)],
                              i_v1, sem3),
    ]
    for cp in cps:
        cp.start()

    def make_group(v_v, i_v, col0):
        def group(g, _):
            row0 = g * 16

            def struct(s):
                return (v_v[s, pl.ds(row0, 16)], i_v[s, pl.ds(row0, 16)],
                        v_v[S + s, pl.ds(row0, 16)], i_v[S + s, pl.ds(row0, 16)])

            m1, a1, m2, a2 = _merge_lex(struct(0), struct(1))

            e2 = jnp.exp(m2 - m1)
            den = 1.0 + e2
            col = col0 + row0
            ob_v[0, pl.ds(col, 16)] = a1.astype(jnp.float32)
            ob_v[1, pl.ds(col, 16)] = a2.astype(jnp.float32)
            ob_v[2, pl.ds(col, 16)] = 1.0 / den
            ob_v[3, pl.ds(col, 16)] = e2 / den
            return 0
        return group

    cps[0].wait()
    cps[1].wait()
    lax.fori_loop(0, GROUPS_H, make_group(v_v0, i_v0, 0), 0)
    cps[2].wait()
    cps[3].wait()
    lax.fori_loop(0, GROUPS_H, make_group(v_v1, i_v1, HALF), 0)

    pltpu.sync_copy(ob_v, out_hbm.at[:, pl.ds(base, ROWS_PER_W)])


_sc_route = pl.kernel(
    _route_body,
    out_type=jax.ShapeDtypeStruct((4, N), jnp.float32),
    mesh=plsc.VectorSubcoreMesh(core_axis_name="c", subcore_axis_name="s"),
    scratch_types=[
        pltpu.VMEM((2 * S, HALF), jnp.float32),
        pltpu.VMEM((2 * S, HALF), jnp.float32),
        pltpu.VMEM((2 * S, HALF), jnp.int32),
        pltpu.VMEM((2 * S, HALF), jnp.int32),
        pltpu.VMEM((4, ROWS_PER_W), jnp.float32),
        pltpu.SemaphoreType.DMA,
        pltpu.SemaphoreType.DMA,
        pltpu.SemaphoreType.DMA,
        pltpu.SemaphoreType.DMA,
    ],
)


@jax.jit
def kernel(x, W):
    vals, idx8 = _tc_partial_top2(x, W)
    out = _sc_route(vals, idx8)
    idx = out[:2].T.astype(jnp.int32)
    gates = out[2:].T
    return idx, gates


# speedup vs baseline: 1.0708x; 1.0708x over previous
import jax
import jax.numpy as jnp
from jax import lax
from jax.experimental import pallas as pl
from jax.experimental.pallas import tpu as pltpu
from jax.experimental.pallas import tpu_sc as plsc

N = 32768
D = 768
E = 64
BLOCK_N = 4096
S = 1

NW = 32
ROWS_PER_W = N // NW
HALF = ROWS_PER_W // 2
GROUPS_H = HALF // 16


def _lex_take(v1, j1, v2, j2):
    return (v1 > v2) | ((v1 == v2) & (j1 < j2))


def _merge_lex(A, B):
    m1a, a1a, m2a, a2a = A
    m1b, a1b, m2b, a2b = B
    take = _lex_take(m1a, a1a, m1b, a1b)
    m1 = jnp.where(take, m1a, m1b)
    a1 = jnp.where(take, a1a, a1b)
    sa = _lex_take(m2a, a2a, m1b, a1b)
    sb = _lex_take(m1a, a1a, m2b, a2b)
    m2 = jnp.where(take, jnp.where(sa, m2a, m1b), jnp.where(sb, m1a, m2b))
    a2 = jnp.where(take, jnp.where(sa, a2a, a1b), jnp.where(sb, a1a, a2b))
    return m1, a1, m2, a2


def _matmul_reduce_body(x_ref, w_ref, vals_ref, idx_ref):
    lt = lax.dot_general(
        w_ref[...], x_ref[...], (((1,), (1,)), ((), ())),
        preferred_element_type=jnp.float32)
    a, b = lt[0:32], lt[32:64]
    ia = jax.lax.broadcasted_iota(jnp.int32, (32, BLOCK_N), 0)
    ib = ia + 32
    take = a >= b
    t = (jnp.maximum(a, b), jnp.where(take, ia, ib),
         jnp.minimum(a, b), jnp.where(take, ib, ia))
    for k in (16, 8, 4, 2, 1):
        t = _merge_lex(tuple(v[:k] for v in t), tuple(v[k:] for v in t))
    m1, i1, m2, i2 = t
    vals_ref[...] = jnp.concatenate([m1, m2], axis=0)
    idx_ref[...] = jnp.concatenate([i1, i2], axis=0)


def _tc_partial_top2(x, W):
    return pl.pallas_call(
        _matmul_reduce_body,
        grid=(N // BLOCK_N,),
        in_specs=[
            pl.BlockSpec((BLOCK_N, D), lambda i: (i, 0)),
            pl.BlockSpec((E, D), lambda i: (0, 0)),
        ],
        out_specs=[
            pl.BlockSpec((2 * S, BLOCK_N), lambda i: (0, i)),
            pl.BlockSpec((2 * S, BLOCK_N), lambda i: (0, i)),
        ],
        out_shape=[
            jax.ShapeDtypeStruct((2 * S, N), jnp.float32),
            jax.ShapeDtypeStruct((2 * S, N), jnp.int32),
        ],
    )(x, W)


def _route_body(vals_hbm, idx_hbm, out_hbm,
                v_v0, v_v1, i_v0, i_v1, ob_v, sem0, sem1, sem2, sem3):
    wid = lax.axis_index("s") * 2 + lax.axis_index("c")
    base = wid * ROWS_PER_W

    cps = [
        pltpu.make_async_copy(vals_hbm.at[:, pl.ds(base, HALF)], v_v0, sem0),
        pltpu.make_async_copy(idx_hbm.at[:, pl.ds(base, HALF)], i_v0, sem1),
        pltpu.make_async_copy(vals_hbm.at[:, pl.ds(base + HALF, HALF)],
                              v_v1, sem2),
        pltpu.make_async_copy(idx_hbm.at[:, pl.ds(base + HALF, HALF)],
                              i_v1, sem3),
    ]
    for cp in cps:
        cp.start()

    def make_group(v_v, i_v, col0):
        def group(g, _):
            row0 = g * 16

            def struct(s):
                return (v_v[s, pl.ds(row0, 16)], i_v[s, pl.ds(row0, 16)],
                        v_v[S + s, pl.ds(row0, 16)], i_v[S + s, pl.ds(row0, 16)])

            m1, a1, m2, a2 = struct(0)

            e2 = jnp.exp(m2 - m1)
            den = 1.0 + e2
            col = col0 + row0
            ob_v[0, pl.ds(col, 16)] = a1.astype(jnp.float32)
            ob_v[1, pl.ds(col, 16)] = a2.astype(jnp.float32)
            ob_v[2, pl.ds(col, 16)] = 1.0 / den
            ob_v[3, pl.ds(col, 16)] = e2 / den
            return 0
        return group

    cps[0].wait()
    cps[1].wait()
    lax.fori_loop(0, GROUPS_H, make_group(v_v0, i_v0, 0), 0)
    cps[2].wait()
    cps[3].wait()
    lax.fori_loop(0, GROUPS_H, make_group(v_v1, i_v1, HALF), 0)

    pltpu.sync_copy(ob_v, out_hbm.at[:, pl.ds(base, ROWS_PER_W)])


_sc_route = pl.kernel(
    _route_body,
    out_type=jax.ShapeDtypeStruct((4, N), jnp.float32),
    mesh=plsc.VectorSubcoreMesh(core_axis_name="c", subcore_axis_name="s"),
    scratch_types=[
        pltpu.VMEM((2 * S, HALF), jnp.float32),
        pltpu.VMEM((2 * S, HALF), jnp.float32),
        pltpu.VMEM((2 * S, HALF), jnp.int32),
        pltpu.VMEM((2 * S, HALF), jnp.int32),
        pltpu.VMEM((4, ROWS_PER_W), jnp.float32),
        pltpu.SemaphoreType.DMA,
        pltpu.SemaphoreType.DMA,
        pltpu.SemaphoreType.DMA,
        pltpu.SemaphoreType.DMA,
    ],
)


@jax.jit
def kernel(x, W):
    vals, idx8 = _tc_partial_top2(x, W)
    out = _sc_route(vals, idx8)
    idx = out[:2].T.astype(jnp.int32)
    gates = out[2:].T
    return idx, gates


# speedup vs baseline: 1.0861x; 1.0143x over previous
import jax
import jax.numpy as jnp
from jax import lax
from jax.experimental import pallas as pl
from jax.experimental.pallas import tpu as pltpu
from jax.experimental.pallas import tpu_sc as plsc

N = 32768
D = 768
E = 64
BLOCK_N = 4096
S = 1

NW = 32
ROWS_PER_W = N // NW
HALF = ROWS_PER_W // 2
GROUPS_H = HALF // 16


def _lex_take(v1, j1, v2, j2):
    return (v1 > v2) | ((v1 == v2) & (j1 < j2))


def _merge_lex(A, B):
    m1a, a1a, m2a, a2a = A
    m1b, a1b, m2b, a2b = B
    take = _lex_take(m1a, a1a, m1b, a1b)
    m1 = jnp.where(take, m1a, m1b)
    a1 = jnp.where(take, a1a, a1b)
    sa = _lex_take(m2a, a2a, m1b, a1b)
    sb = _lex_take(m1a, a1a, m2b, a2b)
    m2 = jnp.where(take, jnp.where(sa, m2a, m1b), jnp.where(sb, m1a, m2b))
    a2 = jnp.where(take, jnp.where(sa, a2a, a1b), jnp.where(sb, a1a, a2b))
    return m1, a1, m2, a2


def _matmul_reduce_body(x_ref, w_ref, packed_ref):
    lt = lax.dot_general(
        w_ref[...], x_ref[...], (((1,), (1,)), ((), ())),
        preferred_element_type=jnp.float32)
    a, b = lt[0:32], lt[32:64]
    ia = jax.lax.broadcasted_iota(jnp.int32, (32, BLOCK_N), 0)
    ib = ia + 32
    take = a >= b
    t = (jnp.maximum(a, b), jnp.where(take, ia, ib),
         jnp.minimum(a, b), jnp.where(take, ib, ia))
    for k in (16, 8, 4, 2, 1):
        t = _merge_lex(tuple(v[:k] for v in t), tuple(v[k:] for v in t))
    m1, i1, m2, i2 = t
    packed_ref[...] = jnp.concatenate([
        m1, m2, (i1 | (i2 << 8)).astype(jnp.float32),
    ], axis=0)


def _tc_partial_top2(x, W):
    return pl.pallas_call(
        _matmul_reduce_body,
        grid=(N // BLOCK_N,),
        in_specs=[
            pl.BlockSpec((BLOCK_N, D), lambda i: (i, 0)),
            pl.BlockSpec((E, D), lambda i: (0, 0)),
        ],
        out_specs=pl.BlockSpec((3, BLOCK_N), lambda i: (0, i)),
        out_shape=jax.ShapeDtypeStruct((3, N), jnp.float32),
    )(x, W)


def _route_body(pk_hbm, out_hbm, pk_v0, pk_v1, ob_v, sem0, sem1):
    wid = lax.axis_index("s") * 2 + lax.axis_index("c")
    base = wid * ROWS_PER_W

    cp0 = pltpu.make_async_copy(pk_hbm.at[:, pl.ds(base, HALF)], pk_v0, sem0)
    cp1 = pltpu.make_async_copy(pk_hbm.at[:, pl.ds(base + HALF, HALF)],
                                pk_v1, sem1)
    cp0.start()
    cp1.start()

    def make_group(pk_v, col0):
        def group(g, _):
            row0 = g * 16
            m1 = pk_v[0, pl.ds(row0, 16)]
            m2 = pk_v[1, pl.ds(row0, 16)]
            packed = pk_v[2, pl.ds(row0, 16)].astype(jnp.int32)
            a1 = packed & 0xFF
            a2 = packed >> 8

            e2 = jnp.exp(m2 - m1)
            den = 1.0 + e2
            col = col0 + row0
            ob_v[0, pl.ds(col, 16)] = a1.astype(jnp.float32)
            ob_v[1, pl.ds(col, 16)] = a2.astype(jnp.float32)
            ob_v[2, pl.ds(col, 16)] = 1.0 / den
            ob_v[3, pl.ds(col, 16)] = e2 / den
            return 0
        return group

    cp0.wait()
    lax.fori_loop(0, GROUPS_H, make_group(pk_v0, 0), 0)
    cp1.wait()
    lax.fori_loop(0, GROUPS_H, make_group(pk_v1, HALF), 0)

    pltpu.sync_copy(ob_v, out_hbm.at[:, pl.ds(base, ROWS_PER_W)])


_sc_route = pl.kernel(
    _route_body,
    out_type=jax.ShapeDtypeStruct((4, N), jnp.float32),
    mesh=plsc.VectorSubcoreMesh(core_axis_name="c", subcore_axis_name="s"),
    scratch_types=[
        pltpu.VMEM((3, HALF), jnp.float32),
        pltpu.VMEM((3, HALF), jnp.float32),
        pltpu.VMEM((4, ROWS_PER_W), jnp.float32),
        pltpu.SemaphoreType.DMA,
        pltpu.SemaphoreType.DMA,
    ],
)


@jax.jit
def kernel(x, W):
    packed = _tc_partial_top2(x, W)
    out = _sc_route(packed)
    idx = out[:2].T.astype(jnp.int32)
    gates = out[2:].T
    return idx, gates
